# baseline (speedup 1.0000x reference)
import jax
import jax.numpy as jnp
from jax.experimental import pallas as pl

_EPS = 1e-5
_BM = 2048


def _mlp_kernel(x_ref, w1_ref, b1_ref, gamma_ref, beta_ref, w2_ref, b2_ref,
                prob_ref, logit_ref):
    x = x_ref[...]
    h = jnp.dot(x, w1_ref[...], preferred_element_type=jnp.float32)
    h = h + b1_ref[...]
    mu = jnp.mean(h, axis=1, keepdims=True)
    d = h - mu
    var = jnp.mean(d * d, axis=1, keepdims=True)
    hn = d * jax.lax.rsqrt(var + _EPS) * gamma_ref[...] + beta_ref[...]
    hr = jnp.maximum(hn, 0.0)
    logits = jnp.sum(hr * w2_ref[...], axis=1) + b2_ref[0]
    logits = jnp.clip(logits, -10.0, 10.0)
    logit_ref[...] = logits
    prob_ref[...] = jax.nn.sigmoid(logits)


def kernel(x, W1, b1, gamma, beta, W2, b2):
    B, T, K = x.shape
    M = B * T
    xf = x.reshape(M, K)
    grid = (M // _BM,)
    probs, logits = pl.pallas_call(
        _mlp_kernel,
        grid=grid,
        in_specs=[
            pl.BlockSpec((_BM, K), lambda i: (i, 0)),
            pl.BlockSpec((K, 256), lambda i: (0, 0)),
            pl.BlockSpec((1, 256), lambda i: (0, 0)),
            pl.BlockSpec((1, 256), lambda i: (0, 0)),
            pl.BlockSpec((1, 256), lambda i: (0, 0)),
            pl.BlockSpec((1, 256), lambda i: (0, 0)),
            pl.BlockSpec((1,), lambda i: (0,)),
        ],
        out_specs=[
            pl.BlockSpec((_BM,), lambda i: (i,)),
            pl.BlockSpec((_BM,), lambda i: (i,)),
        ],
        out_shape=[
            jax.ShapeDtypeStruct((M,), jnp.float32),
            jax.ShapeDtypeStruct((M,), jnp.float32),
        ],
    )(xf, W1, b1.reshape(1, 256), gamma.reshape(1, 256), beta.reshape(1, 256),
      W2.reshape(1, 256), b2)
    return probs.reshape(B, T), logits.reshape(B, T)
